# baseline (speedup 1.0000x reference)
import functools

import jax
import jax.numpy as jnp
from jax import lax
from jax.experimental import pallas as pl
from jax.experimental.pallas import tpu as pltpu
from jax.experimental.pallas import tpu_sc as plsc

B = 16384
NC, NS, L = 2, 16, 16
NW = NC * NS
BPW = B // NW
CHUNKS = BPW // L

_mesh = plsc.VectorSubcoreMesh(core_axis_name="c", subcore_axis_name="s")


@functools.partial(
    pl.kernel,
    mesh=_mesh,
    out_type=jax.ShapeDtypeStruct((B,), jnp.float32),
    compiler_params=pltpu.CompilerParams(
        needs_layout_passes=False,
        skip_device_barrier=True,
        disable_bounds_checks=True,
        disable_semaphore_checks=True,
    ),
    scratch_types=[
        pltpu.VMEM((64,), jnp.float32),
        pltpu.VMEM((BPW,), jnp.int32),
        pltpu.VMEM((BPW,), jnp.int32),
        pltpu.VMEM((BPW,), jnp.int32),
        pltpu.VMEM((BPW,), jnp.int32),
        pltpu.VMEM((BPW,), jnp.int32),
        pltpu.VMEM((BPW,), jnp.float32),
        pltpu.VMEM((BPW,), jnp.float32),
        pltpu.SemaphoreType.DMA,
    ],
)
def _sc_affine_lookup(period, time_ind, cate, cpa_ind, x_segment, x, table,
                      out, tab_v, p_v, t_v, c_v, q_v, s_v, x_v, o_v, sem):
    wid = lax.axis_index("s") * NC + lax.axis_index("c")
    base = wid * BPW
    sl_in = pl.ds(base, BPW)
    copies = [
        pltpu.async_copy(table, tab_v, sem),
        pltpu.async_copy(period.at[sl_in], p_v, sem),
        pltpu.async_copy(time_ind.at[sl_in], t_v, sem),
        pltpu.async_copy(cate.at[sl_in], c_v, sem),
        pltpu.async_copy(cpa_ind.at[sl_in], q_v, sem),
        pltpu.async_copy(x_segment.at[sl_in], s_v, sem),
        pltpu.async_copy(x.at[sl_in], x_v, sem),
    ]
    for c in copies:
        c.wait()
    def chunk(i, carry):
        sl = pl.ds(i * L, L)
        idx = (p_v[sl] * 32 + t_v[sl] * 16 + c_v[sl] * 8
               + q_v[sl] * 4 + s_v[sl] * 2)
        scale = plsc.load_gather(tab_v, [idx])
        shift = plsc.load_gather(tab_v, [idx + 1])
        o_v[sl] = x_v[sl] * scale + shift
        return carry

    half = BPW // 2
    lax.fori_loop(0, CHUNKS // 2, chunk, 0)
    out0 = pltpu.async_copy(
        o_v.at[pl.ds(0, half)], out.at[pl.ds(base, half)], sem)
    lax.fori_loop(CHUNKS // 2, CHUNKS, chunk, 0)
    out1 = pltpu.async_copy(
        o_v.at[pl.ds(half, half)], out.at[pl.ds(base + half, half)], sem)
    out0.wait()
    out1.wait()


def kernel(period, time_ind, cate, cpa_ind, x_segment, x, params):
    table = params.reshape(64).astype(jnp.float32)
    return _sc_affine_lookup(
        period.astype(jnp.int32), time_ind.astype(jnp.int32),
        cate.astype(jnp.int32), cpa_ind.astype(jnp.int32),
        x_segment.astype(jnp.int32), x.astype(jnp.float32), table)


# speedup vs baseline: 1.0140x; 1.0140x over previous
import functools

import jax
import jax.numpy as jnp
from jax import lax
from jax.experimental import pallas as pl
from jax.experimental.pallas import tpu as pltpu
from jax.experimental.pallas import tpu_sc as plsc

B = 16384
NC, NS, L = 2, 16, 16
NW = NC * NS
BPW = B // NW
CHUNKS = BPW // L

_mesh = plsc.VectorSubcoreMesh(core_axis_name="c", subcore_axis_name="s")


@functools.partial(
    pl.kernel,
    mesh=_mesh,
    out_type=jax.ShapeDtypeStruct((B,), jnp.float32),
    compiler_params=pltpu.CompilerParams(
        needs_layout_passes=False,
        skip_device_barrier=True,
        disable_bounds_checks=True,
        disable_semaphore_checks=True,
    ),
    scratch_types=[
        pltpu.VMEM((64,), jnp.float32),
        pltpu.VMEM((BPW,), jnp.int32),
        pltpu.VMEM((BPW,), jnp.int32),
        pltpu.VMEM((BPW,), jnp.int32),
        pltpu.VMEM((BPW,), jnp.int32),
        pltpu.VMEM((BPW,), jnp.int32),
        pltpu.VMEM((BPW,), jnp.float32),
        pltpu.VMEM((BPW,), jnp.float32),
        pltpu.SemaphoreType.DMA,
    ],
)
def _sc_affine_lookup(period, time_ind, cate, cpa_ind, x_segment, x, table,
                      out, tab_v, p_v, t_v, c_v, q_v, s_v, x_v, o_v, sem):
    wid = lax.axis_index("s") * NC + lax.axis_index("c")
    base = wid * BPW
    sl_in = pl.ds(base, BPW)
    copies = [
        pltpu.async_copy(table, tab_v, sem),
        pltpu.async_copy(period.at[sl_in], p_v, sem),
        pltpu.async_copy(time_ind.at[sl_in], t_v, sem),
        pltpu.async_copy(cate.at[sl_in], c_v, sem),
        pltpu.async_copy(cpa_ind.at[sl_in], q_v, sem),
        pltpu.async_copy(x_segment.at[sl_in], s_v, sem),
        pltpu.async_copy(x.at[sl_in], x_v, sem),
    ]
    for c in copies:
        c.wait()
    def chunk(i, carry):
        sl = pl.ds(i * L, L)
        idx = (p_v[sl] * 32 + t_v[sl] * 16 + c_v[sl] * 8
               + q_v[sl] * 4 + s_v[sl] * 2)
        scale = plsc.load_gather(tab_v, [idx])
        shift = plsc.load_gather(tab_v, [idx + 1])
        o_v[sl] = x_v[sl] * scale + shift
        return carry

    lax.fori_loop(0, CHUNKS, chunk, 0)
    pltpu.sync_copy(o_v, out.at[pl.ds(base, BPW)])


def kernel(period, time_ind, cate, cpa_ind, x_segment, x, params):
    table = params.reshape(64).astype(jnp.float32)
    return _sc_affine_lookup(
        period.astype(jnp.int32), time_ind.astype(jnp.int32),
        cate.astype(jnp.int32), cpa_ind.astype(jnp.int32),
        x_segment.astype(jnp.int32), x.astype(jnp.float32), table)


# speedup vs baseline: 1.0259x; 1.0118x over previous
import functools

import jax
import jax.numpy as jnp
from jax import lax
from jax.experimental import pallas as pl
from jax.experimental.pallas import tpu as pltpu
from jax.experimental.pallas import tpu_sc as plsc

B = 16384
NC, NS, L = 2, 16, 16
NW = NC * NS
BPW = B // NW
CHUNKS = BPW // L

_mesh = plsc.VectorSubcoreMesh(core_axis_name="c", subcore_axis_name="s")


@functools.partial(
    pl.kernel,
    mesh=_mesh,
    out_type=jax.ShapeDtypeStruct((B,), jnp.float32),
    compiler_params=pltpu.CompilerParams(
        needs_layout_passes=False,
        skip_device_barrier=True,
        disable_bounds_checks=True,
        disable_semaphore_checks=True,
        use_tc_tiling_on_sc=False,
    ),
    scratch_types=[
        pltpu.VMEM((64,), jnp.float32),
        pltpu.VMEM((BPW,), jnp.int32),
        pltpu.VMEM((BPW,), jnp.int32),
        pltpu.VMEM((BPW,), jnp.int32),
        pltpu.VMEM((BPW,), jnp.int32),
        pltpu.VMEM((BPW,), jnp.int32),
        pltpu.VMEM((BPW,), jnp.float32),
        pltpu.VMEM((BPW,), jnp.float32),
        pltpu.SemaphoreType.DMA,
    ],
)
def _sc_affine_lookup(period, time_ind, cate, cpa_ind, x_segment, x, table,
                      out, tab_v, p_v, t_v, c_v, q_v, s_v, x_v, o_v, sem):
    wid = lax.axis_index("s") * NC + lax.axis_index("c")
    base = wid * BPW
    sl_in = pl.ds(base, BPW)
    copies = [
        pltpu.async_copy(table, tab_v, sem),
        pltpu.async_copy(period.at[sl_in], p_v, sem),
        pltpu.async_copy(time_ind.at[sl_in], t_v, sem),
        pltpu.async_copy(cate.at[sl_in], c_v, sem),
        pltpu.async_copy(cpa_ind.at[sl_in], q_v, sem),
        pltpu.async_copy(x_segment.at[sl_in], s_v, sem),
        pltpu.async_copy(x.at[sl_in], x_v, sem),
    ]
    for c in copies:
        c.wait()
    def chunk(i, carry):
        sl = pl.ds(i * L, L)
        idx = (p_v[sl] * 32 + t_v[sl] * 16 + c_v[sl] * 8
               + q_v[sl] * 4 + s_v[sl] * 2)
        scale = plsc.load_gather(tab_v, [idx])
        shift = plsc.load_gather(tab_v, [idx + 1])
        o_v[sl] = x_v[sl] * scale + shift
        return carry

    lax.fori_loop(0, CHUNKS, chunk, 0)
    pltpu.sync_copy(o_v, out.at[pl.ds(base, BPW)])


def kernel(period, time_ind, cate, cpa_ind, x_segment, x, params):
    table = params.reshape(64).astype(jnp.float32)
    return _sc_affine_lookup(
        period.astype(jnp.int32), time_ind.astype(jnp.int32),
        cate.astype(jnp.int32), cpa_ind.astype(jnp.int32),
        x_segment.astype(jnp.int32), x.astype(jnp.float32), table)


# speedup vs baseline: 1.0263x; 1.0004x over previous
import functools

import jax
import jax.numpy as jnp
from jax import lax
from jax.experimental import pallas as pl
from jax.experimental.pallas import tpu as pltpu
from jax.experimental.pallas import tpu_sc as plsc

B = 16384
NC, NS, L = 2, 16, 16
NW = NC * NS
BPW = B // NW
CHUNKS = BPW // L

_mesh = plsc.VectorSubcoreMesh(core_axis_name="c", subcore_axis_name="s")


@functools.partial(
    pl.kernel,
    mesh=_mesh,
    out_type=jax.ShapeDtypeStruct((B,), jnp.float32),
    compiler_params=pltpu.CompilerParams(
        needs_layout_passes=False,
        skip_device_barrier=True,
        disable_bounds_checks=True,
        disable_semaphore_checks=True,
        use_tc_tiling_on_sc=False,
    ),
    scratch_types=[
        pltpu.VMEM((64,), jnp.float32),
        pltpu.VMEM((BPW,), jnp.int32),
        pltpu.VMEM((BPW,), jnp.int32),
        pltpu.VMEM((BPW,), jnp.int32),
        pltpu.VMEM((BPW,), jnp.int32),
        pltpu.VMEM((BPW,), jnp.int32),
        pltpu.VMEM((BPW,), jnp.float32),
        pltpu.VMEM((BPW,), jnp.float32),
        pltpu.SemaphoreType.DMA,
    ],
)
def _sc_affine_lookup(period, time_ind, cate, cpa_ind, x_segment, x, table,
                      out, tab_v, p_v, t_v, c_v, q_v, s_v, x_v, o_v, sem):
    wid = lax.axis_index("s") * NC + lax.axis_index("c")
    base = wid * BPW
    sl_in = pl.ds(base, BPW)
    copies = [
        pltpu.async_copy(table, tab_v, sem),
        pltpu.async_copy(period.at[sl_in], p_v, sem),
        pltpu.async_copy(time_ind.at[sl_in], t_v, sem),
        pltpu.async_copy(cate.at[sl_in], c_v, sem),
        pltpu.async_copy(cpa_ind.at[sl_in], q_v, sem),
        pltpu.async_copy(x_segment.at[sl_in], s_v, sem),
        pltpu.async_copy(x.at[sl_in], x_v, sem),
    ]
    for c in copies:
        c.wait()
    def chunk(i, carry):
        sl = pl.ds(i * L, L)
        idx = (p_v[sl] * 32 + t_v[sl] * 16 + c_v[sl] * 8
               + q_v[sl] * 4 + s_v[sl] * 2)
        scale = plsc.load_gather(tab_v, [idx])
        shift = plsc.load_gather(tab_v, [idx + 1])
        o_v[sl] = x_v[sl] * scale + shift
        return carry

    half = BPW // 2
    lax.fori_loop(0, CHUNKS // 2, chunk, 0)
    out0 = pltpu.async_copy(
        o_v.at[pl.ds(0, half)], out.at[pl.ds(base, half)], sem)
    lax.fori_loop(CHUNKS // 2, CHUNKS, chunk, 0)
    out1 = pltpu.async_copy(
        o_v.at[pl.ds(half, half)], out.at[pl.ds(base + half, half)], sem)
    out0.wait()
    out1.wait()


def kernel(period, time_ind, cate, cpa_ind, x_segment, x, params):
    table = params.reshape(64).astype(jnp.float32)
    return _sc_affine_lookup(
        period.astype(jnp.int32), time_ind.astype(jnp.int32),
        cate.astype(jnp.int32), cpa_ind.astype(jnp.int32),
        x_segment.astype(jnp.int32), x.astype(jnp.float32), table)
